# baseline (speedup 1.0000x reference)
import functools

import jax
import jax.numpy as jnp
from jax import lax
from jax.experimental import pallas as pl
from jax.experimental.pallas import tpu as pltpu
from jax.experimental.pallas import tpu_sc as plsc

NC = 2
NS = 16
NW = NC * NS
L = 16
CH = 128


@functools.lru_cache(maxsize=None)
def _build(B, V, D):
    b_per_w = B // NW
    n_ch = b_per_w // CH
    mesh = plsc.VectorSubcoreMesh(core_axis_name="c", subcore_axis_name="s")

    @functools.partial(
        pl.kernel,
        mesh=mesh,
        out_type=(
            jax.ShapeDtypeStruct((D, B), jnp.float32),
            jax.ShapeDtypeStruct((D, B), jnp.float32),
        ),
        scratch_types=[
            pltpu.VMEM((b_per_w,), jnp.int32),
            pltpu.VMEM((b_per_w,), jnp.int32),
            pltpu.VMEM((2, b_per_w), jnp.float32),
            pltpu.SemaphoreType.DMA,
        ],
        compiler_params=pltpu.CompilerParams(use_tc_tiling_on_sc=False),
    )
    def k(ori_hbm, dest_hbm, flat_hbm, out_o_hbm, out_d_hbm,
          idx, fidx, vals, sem):
        wid = lax.axis_index("s") * NC + lax.axis_index("c")
        base = wid * b_per_w

        def run(idx_hbm, out_hbm):
            pltpu.sync_copy(idx_hbm.at[pl.ds(base, b_per_w)], idx)

            def fire(j, buf):
                for g in range(b_per_w // L):
                    sl = pl.ds(g * L, L)
                    fidx[sl] = idx[sl] + j * V
                for c in range(n_ch):
                    sl = pl.ds(c * CH, CH)
                    pltpu.async_copy(
                        flat_hbm.at[fidx.at[sl]], vals.at[buf].at[sl], sem)

            def drain():
                for c in range(n_ch):
                    pltpu.make_async_copy(
                        flat_hbm.at[pl.ds(0, CH)],
                        vals.at[0].at[pl.ds(0, CH)], sem).wait()

            fire(0, 0)
            def loop(j, carry):
                buf = lax.rem(j, 2)
                drain()

                @pl.when(j + 1 < D)
                def _():
                    fire(j + 1, 1 - buf)

                pltpu.sync_copy(
                    vals.at[buf], out_hbm.at[j].at[pl.ds(base, b_per_w)])
                return carry

            lax.fori_loop(0, D, loop, 0)

        run(ori_hbm, out_o_hbm)
        run(dest_hbm, out_d_hbm)

    return k


def kernel(ori, dest, table):
    B = ori.shape[0]
    V, D = table.shape
    flat = jax.lax.optimization_barrier(table.T.reshape(V * D))
    out_oT, out_dT = _build(B, V, D)(
        ori.astype(jnp.int32), dest.astype(jnp.int32), flat)
    return out_oT.T, out_dT.T


# speedup vs baseline: 7.2028x; 7.2028x over previous
import functools

import jax
import jax.numpy as jnp
from jax import lax
from jax.experimental import pallas as pl
from jax.experimental.pallas import tpu as pltpu
from jax.experimental.pallas import tpu_sc as plsc

NC = 2
NS = 16
NW = NC * NS
L = 16
CH = 128


@functools.lru_cache(maxsize=None)
def _build(B, D):
    D2 = 2 * D
    b_per_w = B // NW
    n_ch = b_per_w // CH
    mesh = plsc.VectorSubcoreMesh(core_axis_name="c", subcore_axis_name="s")

    @functools.partial(
        pl.kernel,
        mesh=mesh,
        out_type=(
            jax.ShapeDtypeStruct((B // 2, D2), jnp.float32),
            jax.ShapeDtypeStruct((B // 2, D2), jnp.float32),
        ),
        scratch_types=[
            pltpu.VMEM((b_per_w,), jnp.int32),
            pltpu.VMEM((b_per_w,), jnp.int32),
            pltpu.VMEM((2, CH, D2), jnp.float32),
            pltpu.VMEM((CH // 2, D2), jnp.float32),
            pltpu.SemaphoreType.DMA,
        ],
        compiler_params=pltpu.CompilerParams(
            use_tc_tiling_on_sc=True, needs_layout_passes=False),
    )
    def k(ori_hbm, dest_hbm, table2_hbm, out_o_hbm, out_d_hbm,
          idx, pid, stage, sel, sem):
        wid = lax.axis_index("s") * NC + lax.axis_index("c")
        base = wid * b_per_w

        def run(idx_hbm, out_hbm):
            pltpu.sync_copy(idx_hbm.at[pl.ds(base, b_per_w)], idx)
            for g in range(b_per_w // L):
                sl = pl.ds(g * L, L)
                pid[sl] = lax.shift_right_logical(idx[sl], 1)

            def gather(c, buf):
                return pltpu.async_copy(
                    table2_hbm.at[pid.at[pl.ds(c * CH, CH)]],
                    stage.at[buf], sem)

            def select_and_store(c, buf):
                for g in range(CH // L):
                    jv = lax.iota(jnp.int32, L) + (g * L)
                    hv = lax.bitwise_and(idx[pl.ds(c * CH + g * L, L)], 1) * D
                    pv = lax.shift_right_logical(jv, 1)
                    ov = lax.bitwise_and(jv, 1) * D

                    def col(i, carry):
                        cs = jnp.full((L,), i, jnp.int32)
                        vals = plsc.load_gather(
                            stage.at[buf], [jv, hv + cs])
                        plsc.store_scatter(sel, [pv, ov + cs], vals)
                        return carry

                    lax.fori_loop(0, D, col, 0, unroll=4)
                off = pl.multiple_of((base + c * CH) // 2, CH // 2)
                pltpu.sync_copy(sel, out_hbm.at[pl.ds(off, CH // 2)])

            cp = gather(0, 0)
            for c in range(n_ch):
                nxt = gather(c + 1, (c + 1) % 2) if c + 1 < n_ch else None
                cp.wait()
                select_and_store(c, c % 2)
                cp = nxt

        run(ori_hbm, out_o_hbm)
        run(dest_hbm, out_d_hbm)

    return k


def kernel(ori, dest, table):
    B = ori.shape[0]
    V, D = table.shape
    table2 = table.reshape(V // 2, 2 * D)
    out_o2, out_d2 = _build(B, D)(
        ori.astype(jnp.int32), dest.astype(jnp.int32), table2)
    return out_o2.reshape(B, D), out_d2.reshape(B, D)


# speedup vs baseline: 8.0565x; 1.1185x over previous
import functools

import jax
import jax.numpy as jnp
from jax import lax
from jax.experimental import pallas as pl
from jax.experimental.pallas import tpu as pltpu
from jax.experimental.pallas import tpu_sc as plsc

NC = 2
NS = 16
NW = NC * NS
CH = 128


@functools.lru_cache(maxsize=None)
def _build(B, D):
    b_per_w = B // NW
    n_ch = b_per_w // CH
    mesh = plsc.VectorSubcoreMesh(core_axis_name="c", subcore_axis_name="s")

    @functools.partial(
        pl.kernel,
        mesh=mesh,
        out_type=(
            jax.ShapeDtypeStruct((B, D), jnp.float32),
            jax.ShapeDtypeStruct((B, D), jnp.float32),
        ),
        scratch_types=[
            pltpu.VMEM((b_per_w,), jnp.int32),
            pltpu.VMEM((b_per_w,), jnp.int32),
            pltpu.VMEM((b_per_w, D), jnp.float32),
            pltpu.VMEM((b_per_w, D), jnp.float32),
            pltpu.SemaphoreType.DMA,
            pltpu.SemaphoreType.DMA,
        ],
        compiler_params=pltpu.CompilerParams(use_tc_tiling_on_sc=False),
    )
    def k(ori_hbm, dest_hbm, table_hbm, out_o_hbm, out_d_hbm,
          idx_o, idx_d, rows_o, rows_d, sem_o, sem_d):
        wid = lax.axis_index("s") * NC + lax.axis_index("c")
        base = wid * b_per_w
        pltpu.sync_copy(ori_hbm.at[pl.ds(base, b_per_w)], idx_o)
        pltpu.sync_copy(dest_hbm.at[pl.ds(base, b_per_w)], idx_d)
        copies = []
        for j in range(n_ch):
            sl = pl.ds(j * CH, CH)
            copies.append(
                pltpu.async_copy(table_hbm.at[idx_o.at[sl]], rows_o.at[sl], sem_o))
            copies.append(
                pltpu.async_copy(table_hbm.at[idx_d.at[sl]], rows_d.at[sl], sem_d))
        for c in copies:
            c.wait()
        pltpu.sync_copy(rows_o, out_o_hbm.at[pl.ds(base, b_per_w)])
        pltpu.sync_copy(rows_d, out_d_hbm.at[pl.ds(base, b_per_w)])

    return k


def kernel(ori, dest, table):
    B = ori.shape[0]
    D = table.shape[1]
    return _build(B, D)(ori.astype(jnp.int32), dest.astype(jnp.int32), table)


# speedup vs baseline: 13.6465x; 1.6938x over previous
import functools

import jax
import jax.numpy as jnp
from jax import lax
from jax.experimental import pallas as pl
from jax.experimental.pallas import tpu as pltpu
from jax.experimental.pallas import tpu_sc as plsc

NC = 2
NS = 16
NW = NC * NS


@functools.lru_cache(maxsize=None)
def _build(B, D):
    b_per_w = B // NW
    mesh = plsc.VectorSubcoreMesh(core_axis_name="c", subcore_axis_name="s")

    @functools.partial(
        pl.kernel,
        mesh=mesh,
        out_type=(
            jax.ShapeDtypeStruct((B, D), jnp.float32),
            jax.ShapeDtypeStruct((B, D), jnp.float32),
        ),
        scratch_types=[
            pltpu.VMEM((b_per_w,), jnp.int32),
            pltpu.VMEM((b_per_w,), jnp.int32),
            pltpu.VMEM((b_per_w, D), jnp.float32),
            pltpu.SemaphoreType.DMA,
        ],
        compiler_params=pltpu.CompilerParams(
            use_tc_tiling_on_sc=True, needs_layout_passes=False),
    )
    def k(ori_hbm, dest_hbm, table_hbm, out_o_hbm, out_d_hbm,
          idx_o, idx_d, rows, sem):
        wid = lax.axis_index("s") * NC + lax.axis_index("c")
        base = wid * b_per_w
        pltpu.sync_copy(ori_hbm.at[pl.ds(base, b_per_w)], idx_o)
        pltpu.sync_copy(dest_hbm.at[pl.ds(base, b_per_w)], idx_d)

        def run(idx, out_hbm):
            def fire(g, c):
                v = idx[pl.ds(g * 16, 16)]
                for kk in range(16):
                    pltpu.make_async_copy(
                        table_hbm.at[pl.ds(v[kk], 1)],
                        rows.at[pl.ds(g * 16 + kk, 1)], sem).start()
                return c
            lax.fori_loop(0, b_per_w // 16, fire, 0)

            def drain(j, c):
                pltpu.make_async_copy(
                    table_hbm.at[pl.ds(0, 1)],
                    rows.at[pl.ds(j, 1)], sem).wait()
                return c
            lax.fori_loop(0, b_per_w, drain, 0)
            pltpu.sync_copy(rows, out_hbm.at[pl.ds(base, b_per_w)])

        run(idx_o, out_o_hbm)
        run(idx_d, out_d_hbm)

    return k


def kernel(ori, dest, table):
    B = ori.shape[0]
    D = table.shape[1]
    return _build(B, D)(ori.astype(jnp.int32), dest.astype(jnp.int32), table)
